# Initial kernel scaffold; baseline (speedup 1.0000x reference)
import jax
import jax.numpy as jnp
from jax.experimental import pallas as pl


def kernel(x, edge_index, edge_type, W1, root1, b1, W2, root2, b2):
    raise NotImplementedError("write your pallas kernel here")


# speedup vs baseline: 19.3446x; 19.3446x over previous
import functools

import jax
import jax.numpy as jnp
from jax import lax
from jax.experimental import pallas as pl
from jax.experimental.pallas import tpu as pltpu
from jax.experimental.pallas import tpu_sc as plsc

N_NODES = 10000
N_EDGES = 320000
IN_CH = 128
HIDDEN = 64
OUT_CH = 128
NUM_REL = 8

NC, NS, LANES = 2, 16, 16
NW = NC * NS
CNT_PAD = 81920
SLICE = CNT_PAD // NS
EPT = N_EDGES // NW
CH = 2000
CB = 80
NCH = EPT // CB
NPAD = 10240
ZR = 128

@functools.cache
def _mesh():
    return plsc.VectorSubcoreMesh(core_axis_name="c", subcore_axis_name="s",
                                  num_cores=NC, num_subcores=NS)


def _worker_id():
    return lax.axis_index("s") * NC + lax.axis_index("c")


CNT_C = 128
CNT_R = CNT_PAD // CNT_C
_ROWB = CNT_R // 5


@functools.cache
def _cnt_kernel():
    return pl.kernel(
        _cnt_body,
        out_type=jax.ShapeDtypeStruct((NC, CNT_R, CNT_C), jnp.float32),
        mesh=_mesh(),
        compiler_params=pltpu.CompilerParams(needs_layout_passes=False),
        scratch_types=[
            pltpu.VMEM((CNT_R, CNT_C), jnp.float32),
            pltpu.VMEM((CH,), jnp.int32),
            pltpu.VMEM((CH,), jnp.int32),
            pltpu.VMEM((5, _ROWB), jnp.int32),
            pltpu.VMEM_SHARED((CNT_R, CNT_C), jnp.float32),
            pltpu.SemaphoreType.DMA,
        ],
    )


def _cnt_body(dst_hbm, et_hbm, out_hbm, cnt_t, dbuf, tbuf, idxb, acc, sem):
    cid = lax.axis_index("c")
    sid = lax.axis_index("s")
    wid = _worker_id()

    def zero(r, _):
        for q in range(CNT_C // LANES):
            cnt_t[r, pl.ds(q * LANES, LANES)] = jnp.zeros((LANES,), jnp.float32)
        return _

    lax.fori_loop(0, CNT_R, zero, 0)
    rows_per_tile = CNT_R // NS
    pltpu.sync_copy(cnt_t.at[pl.ds(0, rows_per_tile)],
                    acc.at[pl.ds(sid * rows_per_tile, rows_per_tile)])
    for r in range(5):
        for q in range(_ROWB // LANES):
            idxb[r, pl.ds(q * LANES, LANES)] = (
                lax.iota(jnp.int32, LANES) + (r * _ROWB + q * LANES))
    plsc.subcore_barrier()

    ones = jnp.ones((LANES,), jnp.float32)

    def chunk(ci, _):
        off = wid * EPT + ci * CH
        pltpu.sync_copy(dst_hbm.at[pl.ds(off, CH)], dbuf)
        pltpu.sync_copy(et_hbm.at[pl.ds(off, CH)], tbuf)

        def inner(i, carry):
            d = dbuf[pl.ds(i * LANES, LANES)]
            t = tbuf[pl.ds(i * LANES, LANES)]
            kv = d * NUM_REL + t
            plsc.addupdate_scatter(
                cnt_t, [lax.shift_right_logical(kv, 7), kv & (CNT_C - 1)], ones)
            return carry

        return lax.fori_loop(0, CH // LANES, inner, _)

    lax.fori_loop(0, EPT // CH, chunk, 0)

    for r in range(5):
        pltpu.async_copy(cnt_t.at[pl.ds(r * _ROWB, _ROWB)],
                         acc.at[idxb.at[r]], sem, add=True).wait()
    plsc.subcore_barrier()
    r = pl.ds(sid * rows_per_tile, rows_per_tile)
    pltpu.sync_copy(acc.at[r], out_hbm.at[cid, r])


@functools.cache
def _coef_kernel():
    return pl.kernel(
        _coef_body,
        out_type=[
            jax.ShapeDtypeStruct((N_EDGES,), jnp.float32),
            jax.ShapeDtypeStruct((N_EDGES,), jnp.int32),
        ],
        mesh=_mesh(),
        compiler_params=pltpu.CompilerParams(needs_layout_passes=False),
        scratch_types=[
            pltpu.VMEM((CNT_PAD,), jnp.float32),
            pltpu.VMEM((SLICE,), jnp.float32),
            pltpu.VMEM((SLICE,), jnp.float32),
            pltpu.VMEM((CH,), jnp.int32),
            pltpu.VMEM((CH,), jnp.int32),
            pltpu.VMEM((CH,), jnp.int32),
            pltpu.VMEM((CH,), jnp.float32),
            pltpu.VMEM((CH,), jnp.int32),
        ],
    )


def _coef_body(parts_hbm, src_hbm, dst_hbm, et_hbm, c_hbm, g_hbm,
               inv_t, p0, p1, sbuf, dbuf, tbuf, cbuf, gbuf):
    wid = _worker_id()

    for k in range(CNT_PAD // SLICE):
        pltpu.sync_copy(parts_hbm.at[0, pl.ds(k * SLICE, SLICE)], p0)
        pltpu.sync_copy(parts_hbm.at[1, pl.ds(k * SLICE, SLICE)], p1)

        def recip(i, _):
            s = pl.ds(i * LANES, LANES)
            inv_t[pl.ds(k * SLICE + i * LANES, LANES)] = 1.0 / (p0[s] + p1[s])
            return _

        lax.fori_loop(0, SLICE // LANES, recip, 0)

    def chunk(ci, _):
        off = wid * EPT + ci * CH
        pltpu.sync_copy(src_hbm.at[pl.ds(off, CH)], sbuf)
        pltpu.sync_copy(dst_hbm.at[pl.ds(off, CH)], dbuf)
        pltpu.sync_copy(et_hbm.at[pl.ds(off, CH)], tbuf)

        def inner(i, carry):
            s = pl.ds(i * LANES, LANES)
            t = tbuf[s]
            d = dbuf[s]
            cbuf[s] = plsc.load_gather(inv_t, [d * NUM_REL + t])
            gbuf[s] = lax.shift_left(sbuf[s] * NUM_REL + t, 14) | d
            return carry

        lax.fori_loop(0, CH // LANES, inner, _)
        pltpu.sync_copy(cbuf, c_hbm.at[pl.ds(off, CH)])
        pltpu.sync_copy(gbuf, g_hbm.at[pl.ds(off, CH)])
        return _

    lax.fori_loop(0, EPT // CH, chunk, 0)


@functools.cache
def _make_edge_pass(D):
    @functools.partial(
        pl.kernel,
        out_type=jax.ShapeDtypeStruct((NC, NPAD, D), jnp.float32),
        mesh=_mesh(),
        compiler_params=pltpu.CompilerParams(needs_layout_passes=False,
                                             use_tc_tiling_on_sc=False),
        scratch_types=[
            pltpu.VMEM((NCH, CB), jnp.int32),
            pltpu.VMEM((CB,), jnp.int32),
            pltpu.VMEM((CB,), jnp.int32),
            pltpu.VMEM((EPT,), jnp.float32),
            pltpu.VMEM((CB, D), jnp.float32),
            pltpu.VMEM((ZR, D), jnp.float32),
            pltpu.VMEM_SHARED((NPAD, D), jnp.float32),
            pltpu.SemaphoreType.DMA,
            pltpu.SemaphoreType.DMA,
        ],
    )
    def edge_pass(table_hbm, pidx_hbm, c_hbm, out_hbm,
                  pbuf, grow, drow, cbuf, rows, zbuf, acc, sem_g, sem_s):
        cid = lax.axis_index("c")
        sid = lax.axis_index("s")
        wid = _worker_id()

        def zrow(i, _):
            for q in range(D // LANES):
                zbuf[i, pl.ds(q * LANES, LANES)] = jnp.zeros((LANES,), jnp.float32)
            return _

        lax.fori_loop(0, ZR, zrow, 0)
        nrows = NPAD // NS
        for k in range(nrows // ZR):
            pltpu.sync_copy(zbuf, acc.at[pl.ds(sid * nrows + k * ZR, ZR)])
        plsc.subcore_barrier()

        pltpu.sync_copy(pidx_hbm.at[wid], pbuf)
        pltpu.sync_copy(c_hbm.at[pl.ds(wid * EPT, EPT)], cbuf)

        def chunk(j, _):
            for i in range(CB // LANES):
                s = pl.ds(i * LANES, LANES)
                p = pbuf[j, s]
                grow[s] = lax.shift_right_logical(p, 14)
                drow[s] = p & 16383
            pltpu.async_copy(table_hbm.at[grow], rows, sem_g).wait()

            def edge(e, carry):
                cv = plsc.load_gather(
                    cbuf, [jnp.full((LANES,), j * CB + e, jnp.int32)])
                for q in range(D // LANES):
                    s = pl.ds(q * LANES, LANES)
                    rows[e, s] = rows[e, s] * cv
                return carry

            lax.fori_loop(0, CB, edge, 0)
            pltpu.async_copy(rows, acc.at[drow], sem_s, add=True).wait()
            return _

        lax.fori_loop(0, NCH, chunk, 0)
        plsc.subcore_barrier()
        for k in range(nrows // ZR):
            r = pl.ds(sid * nrows + k * ZR, ZR)
            pltpu.sync_copy(acc.at[r], out_hbm.at[cid, r])

    return edge_pass


_RB = 1000


def _mm1_body(x_ref, wc_ref, rt_ref, b_ref, z_ref, xr_ref):
    xb = x_ref[...]
    z_ref[...] = jnp.dot(xb, wc_ref[...], preferred_element_type=jnp.float32)
    xr_ref[...] = (jnp.dot(xb, rt_ref[...], preferred_element_type=jnp.float32)
                   + b_ref[...])


def _mm1(x, wc, rt, b):
    kdim, zdim, rdim = x.shape[1], wc.shape[1], rt.shape[1]
    return pl.pallas_call(
        _mm1_body,
        grid=(N_NODES // _RB,),
        in_specs=[
            pl.BlockSpec((_RB, kdim), lambda i: (i, 0)),
            pl.BlockSpec((kdim, zdim), lambda i: (0, 0)),
            pl.BlockSpec((kdim, rdim), lambda i: (0, 0)),
            pl.BlockSpec((1, rdim), lambda i: (0, 0)),
        ],
        out_specs=[
            pl.BlockSpec((_RB, zdim), lambda i: (i, 0)),
            pl.BlockSpec((_RB, rdim), lambda i: (i, 0)),
        ],
        out_shape=[
            jax.ShapeDtypeStruct((N_NODES, zdim), jnp.float32),
            jax.ShapeDtypeStruct((N_NODES, rdim), jnp.float32),
        ],
    )(x, wc, rt, b)


def _mm2_body(xr_ref, hp_ref, wc_ref, rt_ref, b_ref, z_ref, xr2_ref):
    h = jnp.maximum(xr_ref[...] + hp_ref[0] + hp_ref[1], 0.0)
    z_ref[...] = jnp.dot(h, wc_ref[...], preferred_element_type=jnp.float32)
    xr2_ref[...] = (jnp.dot(h, rt_ref[...], preferred_element_type=jnp.float32)
                    + b_ref[...])


def _mm2(xr, hp, wc, rt, b):
    kdim, zdim, rdim = xr.shape[1], wc.shape[1], rt.shape[1]
    return pl.pallas_call(
        _mm2_body,
        grid=(N_NODES // _RB,),
        in_specs=[
            pl.BlockSpec((_RB, kdim), lambda i: (i, 0)),
            pl.BlockSpec((NC, _RB, kdim), lambda i: (0, i, 0)),
            pl.BlockSpec((kdim, zdim), lambda i: (0, 0)),
            pl.BlockSpec((kdim, rdim), lambda i: (0, 0)),
            pl.BlockSpec((1, rdim), lambda i: (0, 0)),
        ],
        out_specs=[
            pl.BlockSpec((_RB, zdim), lambda i: (i, 0)),
            pl.BlockSpec((_RB, rdim), lambda i: (i, 0)),
        ],
        out_shape=[
            jax.ShapeDtypeStruct((N_NODES, zdim), jnp.float32),
            jax.ShapeDtypeStruct((N_NODES, rdim), jnp.float32),
        ],
    )(xr, hp, wc, rt, b)


def _final_body(xr_ref, hp_ref, o_ref):
    o_ref[...] = xr_ref[...] + hp_ref[0] + hp_ref[1]


def _final(xr, hp):
    d = xr.shape[1]
    return pl.pallas_call(
        _final_body,
        grid=(N_NODES // _RB,),
        in_specs=[
            pl.BlockSpec((_RB, d), lambda i: (i, 0)),
            pl.BlockSpec((NC, _RB, d), lambda i: (0, i, 0)),
        ],
        out_specs=pl.BlockSpec((_RB, d), lambda i: (i, 0)),
        out_shape=jax.ShapeDtypeStruct((N_NODES, d), jnp.float32),
    )(xr, hp)


def kernel(x, edge_index, edge_type, W1, root1, b1, W2, root2, b2):
    src = edge_index[0]
    dst = edge_index[1]
    et = edge_type

    wc1 = jnp.transpose(W1, (1, 0, 2)).reshape(IN_CH, NUM_REL * HIDDEN)
    wc2 = jnp.transpose(W2, (1, 0, 2)).reshape(HIDDEN, NUM_REL * OUT_CH)

    cnt_parts = _cnt_kernel()(dst, et).reshape(NC, CNT_PAD)
    coef, pidx = _coef_kernel()(cnt_parts, src, dst, et)
    pidx3 = pidx.reshape(NW, NCH, CB)

    z1, xr1 = _mm1(x, wc1, root1, b1.reshape(1, HIDDEN))
    h1 = _make_edge_pass(HIDDEN)(
        z1.reshape(N_NODES * NUM_REL, HIDDEN), pidx3, coef)
    z2, xr2 = _mm2(xr1, h1, wc2, root2, b2.reshape(1, OUT_CH))
    h2 = _make_edge_pass(OUT_CH)(
        z2.reshape(N_NODES * NUM_REL, OUT_CH), pidx3, coef)
    return _final(xr2, h2)


# speedup vs baseline: 34.2566x; 1.7709x over previous
import functools

import jax
import jax.numpy as jnp
from jax import lax
from jax.experimental import pallas as pl
from jax.experimental.pallas import tpu as pltpu
from jax.experimental.pallas import tpu_sc as plsc

N_NODES = 10000
N_EDGES = 320000
IN_CH = 128
HIDDEN = 64
OUT_CH = 128
NUM_REL = 8

NC, NS, LANES = 2, 16, 16
NW = NC * NS
CNT_PAD = 81920
SLICE = CNT_PAD // NS
EPT = N_EDGES // NW
CH = 2000
CB = 80
NCH = EPT // CB
NPAD = 10240
ZR = 128

@functools.cache
def _mesh():
    return plsc.VectorSubcoreMesh(core_axis_name="c", subcore_axis_name="s",
                                  num_cores=NC, num_subcores=NS)


def _worker_id():
    return lax.axis_index("s") * NC + lax.axis_index("c")


CNT_C = 128
CNT_R = CNT_PAD // CNT_C
_ROWB = CNT_R // 5


@functools.cache
def _cnt_kernel():
    return pl.kernel(
        _cnt_body,
        out_type=jax.ShapeDtypeStruct((NC, CNT_R, CNT_C), jnp.float32),
        mesh=_mesh(),
        compiler_params=pltpu.CompilerParams(needs_layout_passes=False),
        scratch_types=[
            pltpu.VMEM((CNT_R, CNT_C), jnp.float32),
            pltpu.VMEM((CH,), jnp.int32),
            pltpu.VMEM((CH,), jnp.int32),
            pltpu.VMEM((5, _ROWB), jnp.int32),
            pltpu.VMEM_SHARED((CNT_R, CNT_C), jnp.float32),
            pltpu.SemaphoreType.DMA,
        ],
    )


def _cnt_body(dst_hbm, et_hbm, out_hbm, cnt_t, dbuf, tbuf, idxb, acc, sem):
    cid = lax.axis_index("c")
    sid = lax.axis_index("s")
    wid = _worker_id()

    def zero(r, _):
        for q in range(CNT_C // LANES):
            cnt_t[r, pl.ds(q * LANES, LANES)] = jnp.zeros((LANES,), jnp.float32)
        return _

    lax.fori_loop(0, CNT_R, zero, 0)
    rows_per_tile = CNT_R // NS
    pltpu.sync_copy(cnt_t.at[pl.ds(0, rows_per_tile)],
                    acc.at[pl.ds(sid * rows_per_tile, rows_per_tile)])
    for r in range(5):
        for q in range(_ROWB // LANES):
            idxb[r, pl.ds(q * LANES, LANES)] = (
                lax.iota(jnp.int32, LANES) + (r * _ROWB + q * LANES))
    plsc.subcore_barrier()

    ones = jnp.ones((LANES,), jnp.float32)

    def chunk(ci, _):
        off = wid * EPT + ci * CH
        pltpu.sync_copy(dst_hbm.at[pl.ds(off, CH)], dbuf)
        pltpu.sync_copy(et_hbm.at[pl.ds(off, CH)], tbuf)

        def inner(i, carry):
            d = dbuf[pl.ds(i * LANES, LANES)]
            t = tbuf[pl.ds(i * LANES, LANES)]
            kv = d * NUM_REL + t
            plsc.addupdate_scatter(
                cnt_t, [lax.shift_right_logical(kv, 7), kv & (CNT_C - 1)], ones)
            return carry

        return lax.fori_loop(0, CH // LANES, inner, _)

    lax.fori_loop(0, EPT // CH, chunk, 0)

    for r in range(5):
        pltpu.async_copy(cnt_t.at[pl.ds(r * _ROWB, _ROWB)],
                         acc.at[idxb.at[r]], sem, add=True).wait()
    plsc.subcore_barrier()
    r = pl.ds(sid * rows_per_tile, rows_per_tile)
    pltpu.sync_copy(acc.at[r], out_hbm.at[cid, r])


@functools.cache
def _coef_kernel():
    return pl.kernel(
        _coef_body,
        out_type=[
            jax.ShapeDtypeStruct((N_EDGES,), jnp.float32),
            jax.ShapeDtypeStruct((N_EDGES,), jnp.int32),
        ],
        mesh=_mesh(),
        compiler_params=pltpu.CompilerParams(needs_layout_passes=False),
        scratch_types=[
            pltpu.VMEM((CNT_PAD,), jnp.float32),
            pltpu.VMEM((SLICE,), jnp.float32),
            pltpu.VMEM((SLICE,), jnp.float32),
            pltpu.VMEM((CH,), jnp.int32),
            pltpu.VMEM((CH,), jnp.int32),
            pltpu.VMEM((CH,), jnp.int32),
            pltpu.VMEM((CH,), jnp.float32),
            pltpu.VMEM((CH,), jnp.int32),
        ],
    )


def _coef_body(parts_hbm, src_hbm, dst_hbm, et_hbm, c_hbm, g_hbm,
               cnt_t, p0, p1, sbuf, dbuf, tbuf, cbuf, gbuf):
    wid = _worker_id()

    for k in range(CNT_PAD // SLICE):
        pltpu.sync_copy(parts_hbm.at[0, pl.ds(k * SLICE, SLICE)], p0)
        pltpu.sync_copy(parts_hbm.at[1, pl.ds(k * SLICE, SLICE)], p1)

        @plsc.parallel_loop(0, SLICE // LANES, unroll=4)
        def _sum(i):
            s = pl.ds(i * LANES, LANES)
            cnt_t[pl.ds(k * SLICE + i * LANES, LANES)] = p0[s] + p1[s]

    def chunk(ci, carry):
        off = wid * EPT + ci * CH
        pltpu.sync_copy(src_hbm.at[pl.ds(off, CH)], sbuf)
        pltpu.sync_copy(dst_hbm.at[pl.ds(off, CH)], dbuf)
        pltpu.sync_copy(et_hbm.at[pl.ds(off, CH)], tbuf)

        @plsc.parallel_loop(0, CH // LANES, unroll=4)
        def _edges(i):
            s = pl.ds(i * LANES, LANES)
            t = tbuf[s]
            d = dbuf[s]
            cbuf[s] = 1.0 / plsc.load_gather(cnt_t, [d * NUM_REL + t])
            gbuf[s] = lax.shift_left(sbuf[s] * NUM_REL + t, 14) | d

        pltpu.sync_copy(cbuf, c_hbm.at[pl.ds(off, CH)])
        pltpu.sync_copy(gbuf, g_hbm.at[pl.ds(off, CH)])
        return carry

    lax.fori_loop(0, EPT // CH, chunk, 0)


@functools.cache
def _make_edge_pass(D):
    @functools.partial(
        pl.kernel,
        out_type=jax.ShapeDtypeStruct((NC, NPAD, D), jnp.float32),
        mesh=_mesh(),
        compiler_params=pltpu.CompilerParams(needs_layout_passes=False,
                                             use_tc_tiling_on_sc=False),
        scratch_types=[
            pltpu.VMEM((NCH, CB), jnp.int32),
            pltpu.VMEM((CB,), jnp.int32),
            pltpu.VMEM((CB,), jnp.int32),
            pltpu.VMEM((CB,), jnp.int32),
            pltpu.VMEM((CB,), jnp.int32),
            pltpu.VMEM((EPT,), jnp.float32),
            pltpu.VMEM((CB, D), jnp.float32),
            pltpu.VMEM((CB, D), jnp.float32),
            pltpu.VMEM_SHARED((NPAD, D), jnp.float32),
            pltpu.SemaphoreType.DMA,
            pltpu.SemaphoreType.DMA,
            pltpu.SemaphoreType.DMA,
            pltpu.SemaphoreType.DMA,
        ],
    )
    def edge_pass(table_hbm, pidx_hbm, c_hbm, out_hbm,
                  pbuf, grow0, drow0, grow1, drow1, cbuf, rows0, rows1,
                  acc, sg0, ss0, sg1, ss1):
        cid = lax.axis_index("c")
        sid = lax.axis_index("s")
        wid = _worker_id()

        def zrow(i, _):
            for q in range(D // LANES):
                rows0[i, pl.ds(q * LANES, LANES)] = jnp.zeros((LANES,),
                                                              jnp.float32)
            return _

        lax.fori_loop(0, CB, zrow, 0)
        nrows = NPAD // NS
        for k in range(nrows // CB):
            pltpu.sync_copy(rows0, acc.at[pl.ds(sid * nrows + k * CB, CB)])
        plsc.subcore_barrier()

        pltpu.sync_copy(pidx_hbm.at[wid], pbuf)
        pltpu.sync_copy(c_hbm.at[pl.ds(wid * EPT, EPT)], cbuf)

        def unpack(j, grow, drow):
            for i in range(CB // LANES):
                s = pl.ds(i * LANES, LANES)
                p = pbuf[j, s]
                grow[s] = lax.shift_right_logical(p, 14)
                drow[s] = p & 16383

        def scale(j, rows):
            @plsc.parallel_loop(0, CB, unroll=4)
            def _edge(e):
                cv = plsc.load_gather(
                    cbuf, [jnp.full((LANES,), j * CB + e, jnp.int32)])
                for q in range(D // LANES):
                    s = pl.ds(q * LANES, LANES)
                    rows[e, s] = rows[e, s] * cv

        def step(j, growc, drowc, rowsc, sgc, ssc, grown, drown, rowsn, sgn, ssn):
            @pl.when(j + 1 < NCH)
            def _():
                @pl.when(j >= 1)
                def _():
                    pltpu.make_async_copy(rowsn, acc.at[drown], ssn).wait()
                unpack(j + 1, grown, drown)
                pltpu.async_copy(table_hbm.at[grown], rowsn, sgn)
            pltpu.make_async_copy(table_hbm.at[growc], rowsc, sgc).wait()
            scale(j, rowsc)
            pltpu.async_copy(rowsc, acc.at[drowc], ssc, add=True)

        unpack(0, grow0, drow0)
        pltpu.async_copy(table_hbm.at[grow0], rows0, sg0)

        def chunk(j, carry):
            @pl.when((j & 1) == 0)
            def _even():
                step(j, grow0, drow0, rows0, sg0, ss0, grow1, drow1, rows1, sg1, ss1)

            @pl.when((j & 1) == 1)
            def _odd():
                step(j, grow1, drow1, rows1, sg1, ss1, grow0, drow0, rows0, sg0, ss0)

            return carry

        lax.fori_loop(0, NCH, chunk, 0)
        pltpu.make_async_copy(rows0, acc.at[drow0], ss0).wait()
        pltpu.make_async_copy(rows1, acc.at[drow1], ss1).wait()
        plsc.subcore_barrier()
        for k in range(nrows // ZR):
            r = pl.ds(sid * nrows + k * ZR, ZR)
            pltpu.sync_copy(acc.at[r], out_hbm.at[cid, r])

    return edge_pass


_RB = 1000


def _mm1_body(x_ref, wc_ref, rt_ref, b_ref, z_ref, xr_ref):
    xb = x_ref[...]
    z_ref[...] = jnp.dot(xb, wc_ref[...], preferred_element_type=jnp.float32)
    xr_ref[...] = (jnp.dot(xb, rt_ref[...], preferred_element_type=jnp.float32)
                   + b_ref[...])


def _mm1(x, wc, rt, b):
    kdim, zdim, rdim = x.shape[1], wc.shape[1], rt.shape[1]
    return pl.pallas_call(
        _mm1_body,
        grid=(N_NODES // _RB,),
        in_specs=[
            pl.BlockSpec((_RB, kdim), lambda i: (i, 0)),
            pl.BlockSpec((kdim, zdim), lambda i: (0, 0)),
            pl.BlockSpec((kdim, rdim), lambda i: (0, 0)),
            pl.BlockSpec((1, rdim), lambda i: (0, 0)),
        ],
        out_specs=[
            pl.BlockSpec((_RB, zdim), lambda i: (i, 0)),
            pl.BlockSpec((_RB, rdim), lambda i: (i, 0)),
        ],
        out_shape=[
            jax.ShapeDtypeStruct((N_NODES, zdim), jnp.float32),
            jax.ShapeDtypeStruct((N_NODES, rdim), jnp.float32),
        ],
    )(x, wc, rt, b)


def _mm2_body(xr_ref, hp_ref, wc_ref, rt_ref, b_ref, z_ref, xr2_ref):
    h = jnp.maximum(xr_ref[...] + hp_ref[0] + hp_ref[1], 0.0)
    z_ref[...] = jnp.dot(h, wc_ref[...], preferred_element_type=jnp.float32)
    xr2_ref[...] = (jnp.dot(h, rt_ref[...], preferred_element_type=jnp.float32)
                    + b_ref[...])


def _mm2(xr, hp, wc, rt, b):
    kdim, zdim, rdim = xr.shape[1], wc.shape[1], rt.shape[1]
    return pl.pallas_call(
        _mm2_body,
        grid=(N_NODES // _RB,),
        in_specs=[
            pl.BlockSpec((_RB, kdim), lambda i: (i, 0)),
            pl.BlockSpec((NC, _RB, kdim), lambda i: (0, i, 0)),
            pl.BlockSpec((kdim, zdim), lambda i: (0, 0)),
            pl.BlockSpec((kdim, rdim), lambda i: (0, 0)),
            pl.BlockSpec((1, rdim), lambda i: (0, 0)),
        ],
        out_specs=[
            pl.BlockSpec((_RB, zdim), lambda i: (i, 0)),
            pl.BlockSpec((_RB, rdim), lambda i: (i, 0)),
        ],
        out_shape=[
            jax.ShapeDtypeStruct((N_NODES, zdim), jnp.float32),
            jax.ShapeDtypeStruct((N_NODES, rdim), jnp.float32),
        ],
    )(xr, hp, wc, rt, b)


def _final_body(xr_ref, hp_ref, o_ref):
    o_ref[...] = xr_ref[...] + hp_ref[0] + hp_ref[1]


def _final(xr, hp):
    d = xr.shape[1]
    return pl.pallas_call(
        _final_body,
        grid=(N_NODES // _RB,),
        in_specs=[
            pl.BlockSpec((_RB, d), lambda i: (i, 0)),
            pl.BlockSpec((NC, _RB, d), lambda i: (0, i, 0)),
        ],
        out_specs=pl.BlockSpec((_RB, d), lambda i: (i, 0)),
        out_shape=jax.ShapeDtypeStruct((N_NODES, d), jnp.float32),
    )(xr, hp)


def kernel(x, edge_index, edge_type, W1, root1, b1, W2, root2, b2):
    src = edge_index[0]
    dst = edge_index[1]
    et = edge_type

    wc1 = jnp.transpose(W1, (1, 0, 2)).reshape(IN_CH, NUM_REL * HIDDEN)
    wc2 = jnp.transpose(W2, (1, 0, 2)).reshape(HIDDEN, NUM_REL * OUT_CH)

    cnt_parts = _cnt_kernel()(dst, et).reshape(NC, CNT_PAD)
    coef, pidx = _coef_kernel()(cnt_parts, src, dst, et)
    pidx3 = pidx.reshape(NW, NCH, CB)

    z1, xr1 = _mm1(x, wc1, root1, b1.reshape(1, HIDDEN))
    h1 = _make_edge_pass(HIDDEN)(
        z1.reshape(N_NODES * NUM_REL, HIDDEN), pidx3, coef)
    z2, xr2 = _mm2(xr1, h1, wc2, root2, b2.reshape(1, OUT_CH))
    h2 = _make_edge_pass(OUT_CH)(
        z2.reshape(N_NODES * NUM_REL, OUT_CH), pidx3, coef)
    return _final(xr2, h2)
